# Initial kernel scaffold; baseline (speedup 1.0000x reference)
import jax
import jax.numpy as jnp
from jax.experimental import pallas as pl


def kernel(z, embedding):
    raise NotImplementedError("write your pallas kernel here")


# speedup vs baseline: 1.1802x; 1.1802x over previous
import functools

import jax
import jax.numpy as jnp
from jax import lax
from jax.experimental import pallas as pl
from jax.experimental.pallas import tpu as pltpu
from jax.experimental.pallas import tpu_sc as plsc

_N_E = 1024
_E_DIM = 64
_B = 32
_T = 576
_N_TOK = _B * _T

_BT = 1152
_NT = _N_TOK // _BT

_NW = 32
_BPW = _N_TOK // _NW


def _dist_argmin_body(z_ref, emb_ref, idx_ref, loss_ref):
    i = pl.program_id(0)
    z = z_ref[...]
    emb = emb_ref[...]
    esq = jnp.sum(emb * emb, axis=1)
    zsq = jnp.sum(z * z, axis=1, keepdims=True)
    mm = jnp.dot(z, emb.T, preferred_element_type=jnp.float32)
    d = (zsq + esq[None, :]) - 2.0 * mm
    m = jnp.min(d, axis=1, keepdims=True)
    col = lax.broadcasted_iota(jnp.int32, d.shape, 1)
    idx = jnp.min(jnp.where(d == m, col, _N_E), axis=1)
    idx_ref[0, 0, :] = idx

    @pl.when(i == 0)
    def _init():
        loss_ref[0, 0] = 0.0

    loss_ref[0, 0] += jnp.sum(m)

    @pl.when(i == _NT - 1)
    def _finish():
        loss_ref[0, 0] = loss_ref[0, 0] * (1.25 / float(_N_TOK * _E_DIM))


def _dist_argmin(z_flat, embedding, interpret=False):
    return pl.pallas_call(
        _dist_argmin_body,
        grid=(_NT,),
        in_specs=[
            pl.BlockSpec((_BT, _E_DIM), lambda i: (i, 0)),
            pl.BlockSpec((_N_E, _E_DIM), lambda i: (0, 0)),
        ],
        out_specs=[
            pl.BlockSpec((1, 1, _BT), lambda i: (i, 0, 0)),
            pl.BlockSpec(memory_space=pltpu.SMEM),
        ],
        out_shape=[
            jax.ShapeDtypeStruct((_NT, 1, _BT), jnp.int32),
            jax.ShapeDtypeStruct((1, 1), jnp.float32),
        ],
        interpret=interpret,
    )(z_flat, embedding)


@functools.cache
def _sc_gather_kernel():
    mesh = plsc.VectorSubcoreMesh(core_axis_name="c", subcore_axis_name="s")

    @functools.partial(
        pl.kernel,
        out_type=jax.ShapeDtypeStruct((_N_TOK, _E_DIM), jnp.float32),
        mesh=mesh,
        scratch_types=[
            pltpu.VMEM((_BPW,), jnp.int32),
            pltpu.VMEM((_BPW, _E_DIM), jnp.float32),
            pltpu.SemaphoreType.DMA,
        ],
        compiler_params=pltpu.CompilerParams(use_tc_tiling_on_sc=False),
    )
    def _sc_gather(table_hbm, idx_hbm, out_hbm, idx_v, rows_v, sem):
        wid = lax.axis_index("s") * 2 + lax.axis_index("c")
        base = wid * _BPW
        pltpu.sync_copy(idx_hbm.at[pl.ds(base, _BPW)], idx_v)
        pltpu.async_copy(table_hbm.at[idx_v], rows_v, sem).wait()
        pltpu.sync_copy(rows_v, out_hbm.at[pl.ds(base, _BPW)])

    return _sc_gather


def kernel(z, embedding):
    z_flat = z.reshape(_N_TOK, _E_DIM)
    idx3, loss = _dist_argmin(z_flat, embedding)
    idx = idx3.reshape(_N_TOK)
    z_q = _sc_gather_kernel()(embedding, idx)
    return z_q.reshape(z.shape), idx, loss[0, 0]
